# baseline (speedup 1.0000x reference)
import functools

import jax
import jax.numpy as jnp
from jax import lax
from jax.experimental import pallas as pl
from jax.experimental.pallas import tpu as pltpu
from jax.experimental.pallas import tpu_sc as plsc

VOCAB = 100000
D = 16
F = 26
B = 4096
L = 50
S = F + 1
NSLAB = F + 1
SEQ_BASE = F * VOCAB
NROWS = NSLAB * VOCAB
V_FULL = 99968
TAIL_V = VOCAB - V_FULL
BLK_W = 12 * 128

_INFO = plsc.get_sparse_core_info()
NW = _INFO.num_cores * _INFO.num_subcores
BPW = B // NW
L_CHUNK = 25
INV_L = float(1.0 / L)


def _iota16():
    return lax.iota(jnp.int32, 16)


def _transpose_block(buf, stage, ncols):
    iot16 = _iota16() * 16

    @plsc.parallel_loop(0, ncols // 16, unroll=4)
    def body(jj):
        base = jj * 256
        for d in range(16):
            v = buf[d, pl.ds(jj * 16, 16)]
            plsc.store_scatter(stage, [iot16 + (base + d)], v)


def _repack_body(spT_hbm, sqT_hbm, tail_hbm, out_hbm,
                 buf0, buf1, stage0, stage1, tailbuf,
                 semI0, semI1, semO0, semO1):
    wid = lax.axis_index("s") * _INFO.num_cores + lax.axis_index("c")
    c_base = wid * 24 * 128

    bufs = (buf0, buf1)
    stages = (stage0, stage1)
    semIs = (semI0, semI1)
    semOs = (semO0, semO1)

    def in_cp(item, b):
        s = item // 2
        h = item % 2
        return pltpu.async_copy(
            spT_hbm.at[pl.ds(16 * s, 16), pl.ds(c_base + h * BLK_W, BLK_W)],
            bufs[b], semIs[b])

    def wait_in(b):
        pltpu.make_async_copy(
            spT_hbm.at[pl.ds(0, 16), pl.ds(0, BLK_W)], bufs[b], semIs[b]
        ).wait()

    def wait_out(b):
        pltpu.make_async_copy(
            stages[b], out_hbm.at[pl.ds(0, BLK_W * 16)], semOs[b]).wait()

    def out_off(item):
        s = item // 2
        h = item % 2
        return (s * VOCAB + wid * 24 * 128 + h * BLK_W) * 16

    in_cp(0, 0)
    in_cp(1, 1)

    def step(i, _):
        item = 2 * i
        for b in range(2):
            wait_in(b)

            @pl.when(i > 0)
            def _wo(b=b):
                wait_out(b)

            _transpose_block(bufs[b], stages[b], BLK_W)
            pltpu.async_copy(
                stages[b],
                out_hbm.at[pl.ds(out_off(item + b), BLK_W * 16)], semOs[b])

            @pl.when(i < 25)
            def _f(item=item, b=b):
                in_cp(item + 2 + b, b)

        return _

    lax.fori_loop(0, 26, step, None)
    wait_out(0)
    wait_out(1)

    for h in range(2):
        pltpu.async_copy(
            sqT_hbm.at[:, pl.ds(c_base + h * BLK_W, BLK_W)], buf0, semI0).wait()
        _transpose_block(buf0, stage0, BLK_W)
        pltpu.sync_copy(
            stage0,
            out_hbm.at[pl.ds((SEQ_BASE + wid * 24 * 128 + h * BLK_W) * 16,
                             BLK_W * 16)])

    @pl.when(wid < 13)
    def _extra():
        c0 = (768 + wid) * 128

        def eb(s, _):
            pltpu.async_copy(
                spT_hbm.at[pl.ds(16 * s, 16), pl.ds(c0, 128)],
                buf0.at[:, pl.ds(0, 128)], semI0).wait()
            _transpose_block(buf0, stage0, 128)
            pltpu.sync_copy(
                stage0.at[pl.ds(0, 128 * 16)],
                out_hbm.at[pl.ds((s * VOCAB + c0) * 16, 128 * 16)])
            return _

        lax.fori_loop(0, F, eb, None)
        pltpu.async_copy(sqT_hbm.at[:, pl.ds(c0, 128)],
                         buf0.at[:, pl.ds(0, 128)], semI0).wait()
        _transpose_block(buf0, stage0, 128)
        pltpu.sync_copy(
            stage0.at[pl.ds(0, 128 * 16)],
            out_hbm.at[pl.ds((SEQ_BASE + c0) * 16, 128 * 16)])

    @pl.when(wid < NSLAB)
    def _tail():
        pltpu.sync_copy(tail_hbm, tailbuf)
        iot = _iota16()
        for j in range(TAIL_V):
            w0 = wid * (TAIL_V * 16) + j * 16
            row = plsc.load_gather(
                tailbuf, [w0 // 128 + jnp.zeros((16,), jnp.int32),
                          (w0 % 128) + iot])
            stage0[pl.ds(j * 16, 16)] = row
        base = wid * VOCAB + V_FULL
        pltpu.sync_copy(stage0.at[pl.ds(0, TAIL_V * 16)],
                        out_hbm.at[pl.ds(base * 16, TAIL_V * 16)])


def _gather_body(sidxT_hbm, sqidxT_hbm, tab_hbm, out_hbm,
                 idx_sp, idx_sq, out_blk, seq_rows, semS, semQ):
    wid = lax.axis_index("s") * _INFO.num_cores + lax.axis_index("c")
    col0 = wid * BPW

    pltpu.sync_copy(sidxT_hbm.at[:, pl.ds(col0, BPW)], idx_sp)
    pltpu.sync_copy(sqidxT_hbm.at[:, pl.ds(col0, BPW)], idx_sq)

    for f in range(F):
        for j in range(BPW // 16):
            sl = pl.ds(j * 16, 16)
            idx_sp[f, sl] = idx_sp[f, sl] + (f * VOCAB)
    for l in range(L):
        for j in range(BPW // 16):
            sl = pl.ds(j * 16, 16)
            idx_sq[l, sl] = idx_sq[l, sl] + SEQ_BASE

    sp_cps = []
    for f in range(F):
        sp_cps.append(pltpu.async_copy(
            tab_hbm.at[idx_sp.at[f]],
            out_blk.at[pl.ds(f * BPW, BPW)],
            semS))

    for c in range(2):
        sq_cps = []
        for i in range(L_CHUNK):
            l = c * L_CHUNK + i
            sq_cps.append(pltpu.async_copy(
                tab_hbm.at[idx_sq.at[l]],
                seq_rows.at[pl.ds(i * BPW, BPW)],
                semQ))
        for cp in sq_cps:
            cp.wait()

        def pool_one(bb, _, c=c):
            acc = seq_rows[bb, :]
            for i in range(1, L_CHUNK):
                acc = acc + seq_rows[i * BPW + bb, :]
            r = F * BPW + bb
            if c == 0:
                out_blk[r, :] = acc
            else:
                out_blk[r, :] = (out_blk[r, :] + acc) * INV_L
            return _

        lax.fori_loop(0, BPW, pool_one, None)

    for cp in sp_cps:
        cp.wait()
    pltpu.sync_copy(out_blk, out_hbm.at[pl.ds(wid * S * BPW, S * BPW)])


@functools.partial(jax.jit, static_argnames=())
def kernel(sparse_idx, seq_idx, sparse_tables, seq_table):
    mesh = plsc.VectorSubcoreMesh(core_axis_name="c", subcore_axis_name="s")

    spT = jnp.transpose(sparse_tables, (0, 2, 1)).reshape(F * D, VOCAB)
    sqT = jnp.transpose(seq_table, (1, 0))
    tail128 = jnp.concatenate(
        [sparse_tables[:, V_FULL:, :].reshape(F * TAIL_V, D),
         seq_table[V_FULL:, :]], axis=0).reshape(NSLAB * TAIL_V * D // 128, 128)

    repack = pl.kernel(
        _repack_body,
        out_type=jax.ShapeDtypeStruct((NROWS * D,), jnp.float32),
        mesh=mesh,
        scratch_types=[
            pltpu.VMEM((16, BLK_W), jnp.float32),
            pltpu.VMEM((16, BLK_W), jnp.float32),
            pltpu.VMEM((BLK_W * 16,), jnp.float32),
            pltpu.VMEM((BLK_W * 16,), jnp.float32),
            pltpu.VMEM((NSLAB * TAIL_V * D // 128, 128), jnp.float32),
            pltpu.SemaphoreType.DMA,
            pltpu.SemaphoreType.DMA,
            pltpu.SemaphoreType.DMA,
            pltpu.SemaphoreType.DMA,
        ],
        compiler_params=pltpu.CompilerParams(
            use_tc_tiling_on_sc=True, needs_layout_passes=False),
    )
    tab_flat = repack(spT, sqT, tail128).reshape(NROWS, D)

    sidxT = sparse_idx.T
    sqidxT = seq_idx.T

    gather = pl.kernel(
        _gather_body,
        out_type=jax.ShapeDtypeStruct((NW * S * BPW, D), jnp.float32),
        mesh=mesh,
        scratch_types=[
            pltpu.VMEM((F, BPW), jnp.int32),
            pltpu.VMEM((L, BPW), jnp.int32),
            pltpu.VMEM((S * BPW, D), jnp.float32),
            pltpu.VMEM((L_CHUNK * BPW, D), jnp.float32),
            pltpu.SemaphoreType.DMA,
            pltpu.SemaphoreType.DMA,
        ],
        compiler_params=pltpu.CompilerParams(use_tc_tiling_on_sc=False),
    )
    out = gather(sidxT, sqidxT, tab_flat)
    return out.reshape(NW, S, BPW, D).transpose(0, 2, 1, 3).reshape(B, S, D)


# speedup vs baseline: 1.0890x; 1.0890x over previous
import functools

import jax
import jax.numpy as jnp
from jax import lax
from jax.experimental import pallas as pl
from jax.experimental.pallas import tpu as pltpu
from jax.experimental.pallas import tpu_sc as plsc

VOCAB = 100000
D = 16
F = 26
B = 4096
L = 50
S = F + 1
NSLAB = F + 1
SEQ_BASE = F * VOCAB
NROWS = NSLAB * VOCAB
V_FULL = 99968
TAIL_V = VOCAB - V_FULL
BLK_W = 12 * 128

_INFO = plsc.get_sparse_core_info()
NW = _INFO.num_cores * _INFO.num_subcores
BPW = B // NW
L_CHUNK = 25
INV_L = float(1.0 / L)


def _iota16():
    return lax.iota(jnp.int32, 16)


def _transpose_block(buf, stage, ncols):
    iot16 = _iota16() * 16

    @plsc.parallel_loop(0, ncols // 16, unroll=2)
    def body(jj):
        base = jj * 256
        for d in range(16):
            v = buf[d, pl.ds(jj * 16, 16)]
            plsc.store_scatter(stage, [iot16 + (base + d)], v)


def _repack_body(spT_hbm, sqT_hbm, tail_hbm, out_hbm,
                 buf0, buf1, stage0, stage1, tailbuf,
                 semI0, semI1, semO0, semO1):
    wid = lax.axis_index("s") * _INFO.num_cores + lax.axis_index("c")
    c_base = wid * 24 * 128

    bufs = (buf0, buf1)
    stages = (stage0, stage1)
    semIs = (semI0, semI1)
    semOs = (semO0, semO1)

    def in_cp(item, b):
        s = item // 2
        h = item % 2
        return pltpu.async_copy(
            spT_hbm.at[pl.ds(16 * s, 16), pl.ds(c_base + h * BLK_W, BLK_W)],
            bufs[b], semIs[b])

    def wait_in(b):
        pltpu.make_async_copy(
            spT_hbm.at[pl.ds(0, 16), pl.ds(0, BLK_W)], bufs[b], semIs[b]
        ).wait()

    def wait_out(b):
        pltpu.make_async_copy(
            stages[b], out_hbm.at[pl.ds(0, BLK_W * 16)], semOs[b]).wait()

    def out_off(item):
        s = item // 2
        h = item % 2
        return (s * VOCAB + wid * 24 * 128 + h * BLK_W) * 16

    in_cp(0, 0)
    in_cp(1, 1)

    def step(i, _):
        item = 2 * i
        for b in range(2):
            wait_in(b)

            @pl.when(i > 0)
            def _wo(b=b):
                wait_out(b)

            _transpose_block(bufs[b], stages[b], BLK_W)
            pltpu.async_copy(
                stages[b],
                out_hbm.at[pl.ds(out_off(item + b), BLK_W * 16)], semOs[b])

            @pl.when(i < 25)
            def _f(item=item, b=b):
                in_cp(item + 2 + b, b)

        return _

    lax.fori_loop(0, 26, step, None)
    wait_out(0)
    wait_out(1)

    for h in range(2):
        pltpu.async_copy(
            sqT_hbm.at[:, pl.ds(c_base + h * BLK_W, BLK_W)], buf0, semI0).wait()
        _transpose_block(buf0, stage0, BLK_W)
        pltpu.sync_copy(
            stage0,
            out_hbm.at[pl.ds((SEQ_BASE + wid * 24 * 128 + h * BLK_W) * 16,
                             BLK_W * 16)])

    @pl.when(wid < 13)
    def _extra():
        c0 = (768 + wid) * 128

        def eb(s, _):
            pltpu.async_copy(
                spT_hbm.at[pl.ds(16 * s, 16), pl.ds(c0, 128)],
                buf0.at[:, pl.ds(0, 128)], semI0).wait()
            _transpose_block(buf0, stage0, 128)
            pltpu.sync_copy(
                stage0.at[pl.ds(0, 128 * 16)],
                out_hbm.at[pl.ds((s * VOCAB + c0) * 16, 128 * 16)])
            return _

        lax.fori_loop(0, F, eb, None)
        pltpu.async_copy(sqT_hbm.at[:, pl.ds(c0, 128)],
                         buf0.at[:, pl.ds(0, 128)], semI0).wait()
        _transpose_block(buf0, stage0, 128)
        pltpu.sync_copy(
            stage0.at[pl.ds(0, 128 * 16)],
            out_hbm.at[pl.ds((SEQ_BASE + c0) * 16, 128 * 16)])

    @pl.when(wid < NSLAB)
    def _tail():
        pltpu.sync_copy(tail_hbm, tailbuf)
        iot = _iota16()
        for j in range(TAIL_V):
            w0 = wid * (TAIL_V * 16) + j * 16
            row = plsc.load_gather(
                tailbuf, [w0 // 128 + jnp.zeros((16,), jnp.int32),
                          (w0 % 128) + iot])
            stage0[pl.ds(j * 16, 16)] = row
        base = wid * VOCAB + V_FULL
        pltpu.sync_copy(stage0.at[pl.ds(0, TAIL_V * 16)],
                        out_hbm.at[pl.ds(base * 16, TAIL_V * 16)])


def _gather_body(sidxT_hbm, sqidxT_hbm, tab_hbm, out_hbm,
                 idx_sp, idx_sq, out_blk, seq_rows, semS, semQ):
    wid = lax.axis_index("s") * _INFO.num_cores + lax.axis_index("c")
    col0 = wid * BPW

    pltpu.sync_copy(sidxT_hbm.at[:, pl.ds(col0, BPW)], idx_sp)
    pltpu.sync_copy(sqidxT_hbm.at[:, pl.ds(col0, BPW)], idx_sq)

    for f in range(F):
        for j in range(BPW // 16):
            sl = pl.ds(j * 16, 16)
            idx_sp[f, sl] = idx_sp[f, sl] + (f * VOCAB)
    for l in range(L):
        for j in range(BPW // 16):
            sl = pl.ds(j * 16, 16)
            idx_sq[l, sl] = idx_sq[l, sl] + SEQ_BASE

    sp_cps = []
    for f in range(F):
        sp_cps.append(pltpu.async_copy(
            tab_hbm.at[idx_sp.at[f]],
            out_blk.at[pl.ds(f * BPW, BPW)],
            semS))

    for c in range(2):
        sq_cps = []
        for i in range(L_CHUNK):
            l = c * L_CHUNK + i
            sq_cps.append(pltpu.async_copy(
                tab_hbm.at[idx_sq.at[l]],
                seq_rows.at[pl.ds(i * BPW, BPW)],
                semQ))
        for cp in sq_cps:
            cp.wait()

        def pool_one(bb, _, c=c):
            acc = seq_rows[bb, :]
            for i in range(1, L_CHUNK):
                acc = acc + seq_rows[i * BPW + bb, :]
            r = F * BPW + bb
            if c == 0:
                out_blk[r, :] = acc
            else:
                out_blk[r, :] = (out_blk[r, :] + acc) * INV_L
            return _

        lax.fori_loop(0, BPW, pool_one, None)

    for cp in sp_cps:
        cp.wait()
    pltpu.sync_copy(out_blk, out_hbm.at[pl.ds(wid * S * BPW, S * BPW)])


@functools.partial(jax.jit, static_argnames=())
def kernel(sparse_idx, seq_idx, sparse_tables, seq_table):
    mesh = plsc.VectorSubcoreMesh(core_axis_name="c", subcore_axis_name="s")

    spT = jnp.transpose(sparse_tables, (0, 2, 1)).reshape(F * D, VOCAB)
    sqT = jnp.transpose(seq_table, (1, 0))
    tail128 = jnp.concatenate(
        [sparse_tables[:, V_FULL:, :].reshape(F * TAIL_V, D),
         seq_table[V_FULL:, :]], axis=0).reshape(NSLAB * TAIL_V * D // 128, 128)

    repack = pl.kernel(
        _repack_body,
        out_type=jax.ShapeDtypeStruct((NROWS * D,), jnp.float32),
        mesh=mesh,
        scratch_types=[
            pltpu.VMEM((16, BLK_W), jnp.float32),
            pltpu.VMEM((16, BLK_W), jnp.float32),
            pltpu.VMEM((BLK_W * 16,), jnp.float32),
            pltpu.VMEM((BLK_W * 16,), jnp.float32),
            pltpu.VMEM((NSLAB * TAIL_V * D // 128, 128), jnp.float32),
            pltpu.SemaphoreType.DMA,
            pltpu.SemaphoreType.DMA,
            pltpu.SemaphoreType.DMA,
            pltpu.SemaphoreType.DMA,
        ],
        compiler_params=pltpu.CompilerParams(
            use_tc_tiling_on_sc=True, needs_layout_passes=False),
    )
    tab_flat = repack(spT, sqT, tail128).reshape(NROWS, D)

    sidxT = sparse_idx.T
    sqidxT = seq_idx.T

    gather = pl.kernel(
        _gather_body,
        out_type=jax.ShapeDtypeStruct((NW * S * BPW, D), jnp.float32),
        mesh=mesh,
        scratch_types=[
            pltpu.VMEM((F, BPW), jnp.int32),
            pltpu.VMEM((L, BPW), jnp.int32),
            pltpu.VMEM((S * BPW, D), jnp.float32),
            pltpu.VMEM((L_CHUNK * BPW, D), jnp.float32),
            pltpu.SemaphoreType.DMA,
            pltpu.SemaphoreType.DMA,
        ],
        compiler_params=pltpu.CompilerParams(use_tc_tiling_on_sc=False),
    )
    out = gather(sidxT, sqidxT, tab_flat)
    return out.reshape(NW, S, BPW, D).transpose(0, 2, 1, 3).reshape(B, S, D)
